# baseline (speedup 1.0000x reference)
import jax
import jax.numpy as jnp
from jax import lax
from jax.experimental import pallas as pl
from jax.experimental.pallas import tpu as pltpu
from jax.experimental.pallas import tpu_sc as plsc

N = 10000
E = 320000
F = 128
FH = 64
L = 128
NC = 2
NS = 16
RT = 2560
CW = RT // (NC * NS)
IB = 16
NACC = 10240
DR = NACC // NS
NP = 10240

_MESH = plsc.VectorSubcoreMesh(
    core_axis_name="c", subcore_axis_name="s", num_cores=NC, num_subcores=NS
)


def _hist_body(ei_hbm, out_hbm, acc, idxv, onesv, sem):
    c = lax.axis_index("c")
    s = lax.axis_index("s")
    wid = c * NS + s
    for j in range(8):
        onesv[pl.ds(j * 16, 16)] = jnp.ones((16,), jnp.float32)
    for j in range(40):
        onesv[pl.ds(128 + j * 16, 16)] = jnp.zeros((16,), jnp.float32)
    pltpu.sync_copy(onesv.at[pl.ds(128, 640)], acc.at[pl.ds(s * 640, 640)])
    plsc.subcore_barrier()

    def step(r, carry):
        pltpu.sync_copy(ei_hbm.at[1, r], idxv)
        pltpu.sync_copy(onesv.at[pl.ds(0, L)], acc.at[idxv], add=True)
        return carry

    lax.fori_loop(wid * CW, (wid + 1) * CW, step, 0)
    plsc.subcore_barrier()
    pltpu.sync_copy(acc.at[pl.ds(s * 640, 640)], out_hbm.at[c, pl.ds(s * 640, 640)])


_sc_hist = pl.kernel(
    _hist_body,
    out_type=jax.ShapeDtypeStruct((NC, NP), jnp.float32),
    mesh=_MESH,
    scratch_types=[
        pltpu.VMEM_SHARED((NP,), jnp.float32),
        pltpu.VMEM((L,), jnp.int32),
        pltpu.VMEM((768,), jnp.float32),
        pltpu.SemaphoreType.DMA,
    ],
)


def _zero_acc(s, acc, rows0):
    def zrow(r, carry):
        for j in range(F // 16):
            rows0[r, pl.ds(j * 16, 16)] = jnp.zeros((16,), jnp.float32)
        return carry
    lax.fori_loop(0, L, zrow, 0)
    for k in range(DR // L):
        pltpu.sync_copy(rows0, acc.at[pl.ds(s * DR + k * L, L)])


def _edge_loop(lo, ei_hbm, u2d, srcb, dstb, acc, rows0, rows1, sem0, sem1):
    def batch(b, carry):
        j0 = lo + b * IB
        pltpu.sync_copy(ei_hbm.at[0, pl.ds(j0, IB)], srcb)
        pltpu.sync_copy(ei_hbm.at[1, pl.ds(j0, IB)], dstb)
        pltpu.async_copy(u2d.at[srcb.at[0]], rows0, sem0)

        def pair(p, carry2):
            pltpu.async_copy(u2d.at[srcb.at[2 * p + 1]], rows1, sem1)
            pltpu.make_async_copy(u2d.at[srcb.at[2 * p]], rows0, sem0).wait()
            pltpu.sync_copy(rows0, acc.at[dstb.at[2 * p]], add=True)

            @pl.when(p + 1 < IB // 2)
            def _():
                pltpu.async_copy(u2d.at[srcb.at[2 * p + 2]], rows0, sem0)

            pltpu.make_async_copy(u2d.at[srcb.at[2 * p + 1]], rows1, sem1).wait()
            pltpu.sync_copy(rows1, acc.at[dstb.at[2 * p + 1]], add=True)
            return carry2

        lax.fori_loop(0, IB // 2, pair, 0)
        return carry

    lax.fori_loop(0, CW // IB, batch, 0)


def _prop_b_body(u_hbm, ei_hbm, out_hbm,
                 acc, srcb, dstb, rows0, rows1, sem0, sem1):
    c = lax.axis_index("c")
    s = lax.axis_index("s")
    _zero_acc(s, acc, rows0)
    plsc.subcore_barrier()
    _edge_loop(((1 - c) * NS + s) * CW, ei_hbm, u_hbm, srcb, dstb,
               acc, rows0, rows1, sem0, sem1)
    plsc.subcore_barrier()
    pltpu.sync_copy(acc.at[pl.ds(s * DR, DR)], out_hbm.at[c, pl.ds(s * DR, DR)])


_sc_prop_b = pl.kernel(
    _prop_b_body,
    out_type=jax.ShapeDtypeStruct((NC, NACC, F), jnp.float32),
    mesh=_MESH,
    scratch_types=[
        pltpu.VMEM_SHARED((NACC, F), jnp.float32),
        pltpu.VMEM((IB, L), jnp.int32),
        pltpu.VMEM((IB, L), jnp.int32),
        pltpu.VMEM((L, F), jnp.float32),
        pltpu.VMEM((L, F), jnp.float32),
        pltpu.SemaphoreType.DMA,
        pltpu.SemaphoreType.DMA,
    ],
)


def _prop_c_body(u_hbm, ei_hbm, out_hbm,
                 acc, srcb, dstb, rows0, rows1, sem0, sem1):
    c = lax.axis_index("c")
    s = lax.axis_index("s")
    for h in range(2):
        _zero_acc(s, acc, rows0)
        plsc.subcore_barrier()
        _edge_loop((c * NS + s) * CW, ei_hbm, u_hbm.at[h], srcb, dstb,
                   acc, rows0, rows1, sem0, sem1)
        plsc.subcore_barrier()
        pltpu.sync_copy(
            acc.at[pl.ds(s * DR, DR)], out_hbm.at[h, c, pl.ds(s * DR, DR)])


_sc_prop_c = pl.kernel(
    _prop_c_body,
    out_type=jax.ShapeDtypeStruct((2, NC, NACC, F), jnp.float32),
    mesh=_MESH,
    scratch_types=[
        pltpu.VMEM_SHARED((NACC, F), jnp.float32),
        pltpu.VMEM((IB, L), jnp.int32),
        pltpu.VMEM((IB, L), jnp.int32),
        pltpu.VMEM((L, F), jnp.float32),
        pltpu.VMEM((L, F), jnp.float32),
        pltpu.SemaphoreType.DMA,
        pltpu.SemaphoreType.DMA,
    ],
)


_BLK = 1000
_GRID = N // _BLK


def _tc1_body(x_ref, hist_ref, w1_ref, b1_ref,
              y1_ref, u1_ref, invs_ref, invd_ref):
    cnt = hist_ref[0] + hist_ref[1]
    deg = cnt + 1.0
    invs = lax.rsqrt(deg)
    invd = 1.0 / deg
    y1 = jnp.dot(x_ref[...], w1_ref[...],
                 preferred_element_type=jnp.float32) + b1_ref[...]
    y1_ref[...] = y1
    u1_ref[...] = invs * y1
    invs_ref[...] = invs
    invd_ref[...] = invd


def _tc2_body(mp_ref, y1_ref, invs_ref, invd_ref, w2_ref, b2_ref,
              y2_ref, u2p_ref):
    invs = invs_ref[...]
    invd = invd_ref[...]
    y1 = y1_ref[...]
    m = invs * (mp_ref[0] + mp_ref[1])
    h_lp = jnp.maximum(m + invd * y1, 0.0)
    h_hp = jnp.maximum(y1 - m, 0.0)
    w2 = w2_ref[...]
    b2 = b2_ref[...]
    y2_lp = jnp.dot(h_lp, w2, preferred_element_type=jnp.float32) + b2
    y2_hp = jnp.dot(h_hp, w2, preferred_element_type=jnp.float32) + b2
    y2_ref[0] = y2_lp
    y2_ref[1] = y2_hp
    u2_lp = invs * y2_lp
    u2_hp = invs * y2_hp
    u2p_ref[0] = jnp.concatenate([u2_lp[:, :FH], u2_hp[:, :FH]], axis=-1)
    u2p_ref[1] = jnp.concatenate([u2_lp[:, FH:], u2_hp[:, FH:]], axis=-1)


def _tc3_body(zp_ref, y2_ref, invs_ref, invd_ref, w3_ref, b3_ref,
              zlp_ref, zhp_ref, plp_ref, php_ref):
    invs = invs_ref[...]
    invd = invd_ref[...]
    ph0 = zp_ref[0, 0] + zp_ref[0, 1]
    ph1 = zp_ref[1, 0] + zp_ref[1, 1]
    a_lp = jnp.concatenate([ph0[:, :FH], ph1[:, :FH]], axis=-1)
    a_hp = jnp.concatenate([ph0[:, FH:], ph1[:, FH:]], axis=-1)
    z_lp = invs * a_lp + invd * y2_ref[0]
    z_hp = y2_ref[1] - invs * a_hp
    w3 = w3_ref[...]
    b3 = b3_ref[...]
    zlp_ref[...] = z_lp
    zhp_ref[...] = z_hp
    plp_ref[...] = jnp.dot(z_lp, w3, preferred_element_type=jnp.float32) + b3
    php_ref[...] = jnp.dot(z_hp, w3, preferred_element_type=jnp.float32) + b3


def _rows(i):
    return (i, 0)


def _full(i):
    return (0, 0)


def _rows3(i):
    return (0, i, 0)


def _rows4(i):
    return (0, 0, i, 0)


_spec_nf = pl.BlockSpec((_BLK, F), _rows)
_spec_n1 = pl.BlockSpec((_BLK, 1), _rows)
_spec_2n1 = pl.BlockSpec((NC, _BLK, 1), _rows3)
_spec_2nf = pl.BlockSpec((NC, _BLK, F), _rows3)
_spec_4nf = pl.BlockSpec((2, NC, _BLK, F), _rows4)
_spec_w = pl.BlockSpec((F, F), _full)
_spec_b = pl.BlockSpec((1, F), _full)

_nf = jax.ShapeDtypeStruct((N, F), jnp.float32)
_n1 = jax.ShapeDtypeStruct((N, 1), jnp.float32)
_2nf = jax.ShapeDtypeStruct((NC, N, F), jnp.float32)

_tc1 = pl.pallas_call(
    _tc1_body,
    grid=(_GRID,),
    in_specs=[_spec_nf, _spec_2n1, _spec_w, _spec_b],
    out_specs=[_spec_nf, _spec_nf, _spec_n1, _spec_n1],
    out_shape=[_nf, _nf, _n1, _n1],
)

_tc2 = pl.pallas_call(
    _tc2_body,
    grid=(_GRID,),
    in_specs=[_spec_2nf, _spec_nf, _spec_n1, _spec_n1, _spec_w, _spec_b],
    out_specs=[_spec_2nf, _spec_2nf],
    out_shape=[_2nf, _2nf],
)

_tc3 = pl.pallas_call(
    _tc3_body,
    grid=(_GRID,),
    in_specs=[_spec_4nf, _spec_2nf, _spec_n1, _spec_n1, _spec_w, _spec_b],
    out_specs=[_spec_nf, _spec_nf, _spec_nf, _spec_nf],
    out_shape=[_nf, _nf, _nf, _nf],
)


@jax.jit
def kernel(x, edge_index, W1, b1, W2, b2, W3, b3):
    pad = RT * L - E
    src = jnp.concatenate([edge_index[0], jnp.zeros((pad,), jnp.int32)])
    dst = jnp.concatenate(
        [edge_index[1], N + (jnp.arange(pad, dtype=jnp.int32) % (NACC - N))])
    ei = jnp.stack([src, dst]).reshape(2, RT, L)

    histp = _sc_hist(ei)
    hist2 = histp[:, :N].reshape(NC, N, 1)

    b1r = b1.reshape(1, F)
    b2r = b2.reshape(1, F)
    b3r = b3.reshape(1, F)

    y1, u1, invs, invd = _tc1(x, hist2, W1, b1r)
    mp = _sc_prop_b(u1, ei)
    y2, u2p = _tc2(mp, y1, invs, invd, W2, b2r)
    zp = _sc_prop_c(u2p, ei)
    z_lp, z_hp, p_lp, p_hp = _tc3(zp, y2, invs, invd, W3, b3r)
    return (z_lp, z_hp, p_lp, p_hp, p_lp, p_hp)


# speedup vs baseline: 2.8268x; 2.8268x over previous
import jax
import jax.numpy as jnp
from jax import lax
from jax.experimental import pallas as pl
from jax.experimental.pallas import tpu as pltpu
from jax.experimental.pallas import tpu_sc as plsc

N = 10000
E = 320000
F = 128
FH = 64
L = 128
NC = 2
NS = 16
RT = 2560
CW = RT // (NC * NS)
IB = 16
NACC = 10240
DR = NACC // NS
NP = 10240
Z = 640

_MESH = plsc.VectorSubcoreMesh(
    core_axis_name="c", subcore_axis_name="s", num_cores=NC, num_subcores=NS
)


def _hist_body(dst_hbm, out_hbm, acc, idxv, onesv, sem):
    c = lax.axis_index("c")
    s = lax.axis_index("s")
    wid = c * NS + s
    for j in range(8):
        onesv[pl.ds(j * 16, 16)] = jnp.ones((16,), jnp.float32)
    for j in range(40):
        onesv[pl.ds(128 + j * 16, 16)] = jnp.zeros((16,), jnp.float32)
    pltpu.sync_copy(onesv.at[pl.ds(128, 640)], acc.at[pl.ds(s * 640, 640)])
    plsc.subcore_barrier()

    def step(r, carry):
        pltpu.sync_copy(dst_hbm.at[r], idxv)
        pltpu.sync_copy(onesv.at[pl.ds(0, L)], acc.at[idxv], add=True)
        return carry

    lax.fori_loop(wid * CW, (wid + 1) * CW, step, 0)
    plsc.subcore_barrier()
    pltpu.sync_copy(acc.at[pl.ds(s * 640, 640)], out_hbm.at[c, pl.ds(s * 640, 640)])


_sc_hist = pl.kernel(
    _hist_body,
    out_type=jax.ShapeDtypeStruct((NC, NP), jnp.float32),
    mesh=_MESH,
    scratch_types=[
        pltpu.VMEM_SHARED((NP,), jnp.float32),
        pltpu.VMEM((L,), jnp.int32),
        pltpu.VMEM((768,), jnp.float32),
        pltpu.SemaphoreType.DMA,
    ],
)


def _zero_acc(s, acc, rows0):
    def zrow(r, carry):
        for j in range(F // 16):
            rows0[r, pl.ds(j * 16, 16)] = jnp.zeros((16,), jnp.float32)
        return carry
    lax.fori_loop(0, L, zrow, 0)
    for k in range(DR // L):
        pltpu.sync_copy(rows0, acc.at[pl.ds(s * DR + k * L, L)])


def _edge_loop(lo, ei_hbm, u2d, srcb, dstb, acc, rows0, rows1, sem0, sem1):
    def batch(b, carry):
        j0 = lo + b * IB
        pltpu.sync_copy(ei_hbm.at[0, pl.ds(j0, IB)], srcb)
        pltpu.sync_copy(ei_hbm.at[1, pl.ds(j0, IB)], dstb)
        pltpu.async_copy(u2d.at[srcb.at[0]], rows0, sem0)

        def pair(p, carry2):
            pltpu.async_copy(u2d.at[srcb.at[2 * p + 1]], rows1, sem1)
            pltpu.make_async_copy(u2d.at[srcb.at[2 * p]], rows0, sem0).wait()
            pltpu.sync_copy(rows0, acc.at[dstb.at[2 * p]], add=True)

            @pl.when(p + 1 < IB // 2)
            def _():
                pltpu.async_copy(u2d.at[srcb.at[2 * p + 2]], rows0, sem0)

            pltpu.make_async_copy(u2d.at[srcb.at[2 * p + 1]], rows1, sem1).wait()
            pltpu.sync_copy(rows1, acc.at[dstb.at[2 * p + 1]], add=True)
            return carry2

        lax.fori_loop(0, IB // 2, pair, 0)
        return carry

    lax.fori_loop(0, CW // IB, batch, 0)


def _prop_b_body(u_hbm, ei_hbm, out_hbm,
                 acc, srcb, dstb, rows0, rows1, sem0, sem1):
    c = lax.axis_index("c")
    s = lax.axis_index("s")
    _zero_acc(s, acc, rows0)
    plsc.subcore_barrier()
    _edge_loop((c * NS + s) * CW, ei_hbm, u_hbm, srcb, dstb,
               acc, rows0, rows1, sem0, sem1)
    plsc.subcore_barrier()
    pltpu.sync_copy(acc.at[pl.ds(s * DR, DR)], out_hbm.at[c, pl.ds(s * DR, DR)])


_sc_prop_b = pl.kernel(
    _prop_b_body,
    out_type=jax.ShapeDtypeStruct((NC, NACC, F), jnp.float32),
    mesh=_MESH,
    scratch_types=[
        pltpu.VMEM_SHARED((NACC, F), jnp.float32),
        pltpu.VMEM((IB, L), jnp.int32),
        pltpu.VMEM((IB, L), jnp.int32),
        pltpu.VMEM((L, F), jnp.float32),
        pltpu.VMEM((L, F), jnp.float32),
        pltpu.SemaphoreType.DMA,
        pltpu.SemaphoreType.DMA,
    ],
)


def _prop_c_body(u_hbm, ei_hbm, out_hbm,
                 acc, srcb, dstb, rows0, rows1, sem0, sem1):
    c = lax.axis_index("c")
    s = lax.axis_index("s")
    for h in range(2):
        _zero_acc(s, acc, rows0)
        plsc.subcore_barrier()
        _edge_loop((c * NS + s) * CW, ei_hbm, u_hbm.at[h], srcb, dstb,
                   acc, rows0, rows1, sem0, sem1)
        plsc.subcore_barrier()
        pltpu.sync_copy(
            acc.at[pl.ds(s * DR, DR)], out_hbm.at[h, c, pl.ds(s * DR, DR)])


_sc_prop_c = pl.kernel(
    _prop_c_body,
    out_type=jax.ShapeDtypeStruct((2, NC, NACC, F), jnp.float32),
    mesh=_MESH,
    scratch_types=[
        pltpu.VMEM_SHARED((NACC, F), jnp.float32),
        pltpu.VMEM((IB, L), jnp.int32),
        pltpu.VMEM((IB, L), jnp.int32),
        pltpu.VMEM((L, F), jnp.float32),
        pltpu.VMEM((L, F), jnp.float32),
        pltpu.SemaphoreType.DMA,
        pltpu.SemaphoreType.DMA,
    ],
)


_BLK = 1000
_GRID = N // _BLK


def _tc1_body(x_ref, hist_ref, w1_ref, b1_ref,
              y1_ref, u1_ref, invs_ref, invd_ref):
    cnt = hist_ref[0] + hist_ref[1]
    deg = cnt + 1.0
    invs = lax.rsqrt(deg)
    invd = 1.0 / deg
    y1 = jnp.dot(x_ref[...], w1_ref[...],
                 preferred_element_type=jnp.float32) + b1_ref[...]
    y1_ref[...] = y1
    u1_ref[...] = invs * y1
    invs_ref[...] = invs
    invd_ref[...] = invd


def _tc2_body(mp_ref, y1_ref, invs_ref, invd_ref, w2_ref, b2_ref,
              y2_ref, u2p_ref):
    invs = invs_ref[...]
    invd = invd_ref[...]
    y1 = y1_ref[...]
    m = invs * (mp_ref[0] + mp_ref[1])
    h_lp = jnp.maximum(m + invd * y1, 0.0)
    h_hp = jnp.maximum(y1 - m, 0.0)
    w2 = w2_ref[...]
    b2 = b2_ref[...]
    y2_lp = jnp.dot(h_lp, w2, preferred_element_type=jnp.float32) + b2
    y2_hp = jnp.dot(h_hp, w2, preferred_element_type=jnp.float32) + b2
    y2_ref[0] = y2_lp
    y2_ref[1] = y2_hp
    u2_lp = invs * y2_lp
    u2_hp = invs * y2_hp
    u2p_ref[0] = jnp.concatenate([u2_lp[:, :FH], u2_hp[:, :FH]], axis=-1)
    u2p_ref[1] = jnp.concatenate([u2_lp[:, FH:], u2_hp[:, FH:]], axis=-1)


def _tc3_body(zp_ref, y2_ref, invs_ref, invd_ref, w3_ref, b3_ref,
              zlp_ref, zhp_ref, plp_ref, php_ref):
    invs = invs_ref[...]
    invd = invd_ref[...]
    ph0 = zp_ref[0, 0] + zp_ref[0, 1]
    ph1 = zp_ref[1, 0] + zp_ref[1, 1]
    a_lp = jnp.concatenate([ph0[:, :FH], ph1[:, :FH]], axis=-1)
    a_hp = jnp.concatenate([ph0[:, FH:], ph1[:, FH:]], axis=-1)
    z_lp = invs * a_lp + invd * y2_ref[0]
    z_hp = y2_ref[1] - invs * a_hp
    w3 = w3_ref[...]
    b3 = b3_ref[...]
    zlp_ref[...] = z_lp
    zhp_ref[...] = z_hp
    plp_ref[...] = jnp.dot(z_lp, w3, preferred_element_type=jnp.float32) + b3
    php_ref[...] = jnp.dot(z_hp, w3, preferred_element_type=jnp.float32) + b3


def _rows(i):
    return (i, 0)


def _full(i):
    return (0, 0)


def _rows3(i):
    return (0, i, 0)


def _rows4(i):
    return (0, 0, i, 0)


_spec_nf = pl.BlockSpec((_BLK, F), _rows)
_spec_n1 = pl.BlockSpec((_BLK, 1), _rows)
_spec_2n1 = pl.BlockSpec((NC, _BLK, 1), _rows3)
_spec_2nf = pl.BlockSpec((NC, _BLK, F), _rows3)
_spec_4nf = pl.BlockSpec((2, NC, _BLK, F), _rows4)
_spec_w = pl.BlockSpec((F, F), _full)
_spec_b = pl.BlockSpec((1, F), _full)

_nf = jax.ShapeDtypeStruct((N, F), jnp.float32)
_n1 = jax.ShapeDtypeStruct((N, 1), jnp.float32)
_2nf = jax.ShapeDtypeStruct((NC, N, F), jnp.float32)

_tc1 = pl.pallas_call(
    _tc1_body,
    grid=(_GRID,),
    in_specs=[_spec_nf, _spec_2n1, _spec_w, _spec_b],
    out_specs=[_spec_nf, _spec_nf, _spec_n1, _spec_n1],
    out_shape=[_nf, _nf, _n1, _n1],
)

_tc2 = pl.pallas_call(
    _tc2_body,
    grid=(_GRID,),
    in_specs=[_spec_2nf, _spec_nf, _spec_n1, _spec_n1, _spec_w, _spec_b],
    out_specs=[_spec_2nf, _spec_2nf],
    out_shape=[_2nf, _2nf],
)

_tc3 = pl.pallas_call(
    _tc3_body,
    grid=(_GRID,),
    in_specs=[_spec_4nf, _spec_2nf, _spec_n1, _spec_n1, _spec_w, _spec_b],
    out_specs=[_spec_nf, _spec_nf, _spec_nf, _spec_nf],
    out_shape=[_nf, _nf, _nf, _nf],
)


@jax.jit
def kernel(x, edge_index, W1, b1, W2, b2, W3, b3):
    pad = RT * L - E
    ip = jnp.arange(pad, dtype=jnp.int32)
    src = jnp.concatenate([edge_index[0], N + (ip % Z)])
    dst = jnp.concatenate(
        [edge_index[1], (ip % NS) * DR + (ip // NS) % DR])
    ei = jnp.stack([src, dst]).reshape(2, RT, L)
    dst_hist = jnp.concatenate(
        [edge_index[1], N + (ip % (NP - N))]).reshape(RT, L)

    histp = _sc_hist(dst_hist)
    hist2 = histp[:, :N].reshape(NC, N, 1)

    b1r = b1.reshape(1, F)
    b2r = b2.reshape(1, F)
    b3r = b3.reshape(1, F)

    y1, u1, invs, invd = _tc1(x, hist2, W1, b1r)
    u1z = jnp.concatenate([u1, jnp.zeros((Z, F), jnp.float32)])
    mp = _sc_prop_b(u1z, ei)
    y2, u2p = _tc2(mp, y1, invs, invd, W2, b2r)
    u2pz = jnp.concatenate(
        [u2p, jnp.zeros((NC, Z, F), jnp.float32)], axis=1)
    zp = _sc_prop_c(u2pz, ei)
    z_lp, z_hp, p_lp, p_hp = _tc3(zp, y2, invs, invd, W3, b3r)
    return (z_lp, z_hp, p_lp, p_hp, p_lp, p_hp)


# speedup vs baseline: 2.9706x; 1.0509x over previous
import jax
import jax.numpy as jnp
from jax import lax
from jax.experimental import pallas as pl
from jax.experimental.pallas import tpu as pltpu
from jax.experimental.pallas import tpu_sc as plsc

N = 10000
E = 320000
F = 128
FH = 64
L = 128
NC = 2
NS = 16
RT = 2560
CW = RT // (NC * NS)
NACC = 10240
DR = NACC // NS
NP = 10240
Z = 640

_MESH = plsc.VectorSubcoreMesh(
    core_axis_name="c", subcore_axis_name="s", num_cores=NC, num_subcores=NS
)


def _hist_body(dst_hbm, out_hbm, acc, idxv, onesv, sem):
    c = lax.axis_index("c")
    s = lax.axis_index("s")
    wid = c * NS + s
    for j in range(8):
        onesv[pl.ds(j * 16, 16)] = jnp.ones((16,), jnp.float32)
    for j in range(40):
        onesv[pl.ds(128 + j * 16, 16)] = jnp.zeros((16,), jnp.float32)
    pltpu.sync_copy(onesv.at[pl.ds(128, 640)], acc.at[pl.ds(s * 640, 640)])
    plsc.subcore_barrier()

    def step(r, carry):
        pltpu.sync_copy(dst_hbm.at[r], idxv)
        pltpu.sync_copy(onesv.at[pl.ds(0, L)], acc.at[idxv], add=True)
        return carry

    lax.fori_loop(wid * CW, (wid + 1) * CW, step, 0)
    plsc.subcore_barrier()
    pltpu.sync_copy(acc.at[pl.ds(s * 640, 640)], out_hbm.at[c, pl.ds(s * 640, 640)])


_sc_hist = pl.kernel(
    _hist_body,
    out_type=jax.ShapeDtypeStruct((NC, NP), jnp.float32),
    mesh=_MESH,
    scratch_types=[
        pltpu.VMEM_SHARED((NP,), jnp.float32),
        pltpu.VMEM((L,), jnp.int32),
        pltpu.VMEM((768,), jnp.float32),
        pltpu.SemaphoreType.DMA,
    ],
)


def _zero_acc(s, acc, rows0):
    def zrow(r, carry):
        for j in range(F // 16):
            rows0[r, pl.ds(j * 16, 16)] = jnp.zeros((16,), jnp.float32)
        return carry
    lax.fori_loop(0, L, zrow, 0)
    for k in range(DR // L):
        pltpu.sync_copy(rows0, acc.at[pl.ds(s * DR + k * L, L)])


BST = 40


def _edge_loop(lo, ei_hbm, u2d, srcb, dstb, acc, b0, b1, g0, g1, s0, s1):
    def batch(bi, carry):
        base = lo + bi * BST
        pltpu.sync_copy(ei_hbm.at[0, pl.ds(base, BST)], srcb)
        pltpu.sync_copy(ei_hbm.at[1, pl.ds(base, BST)], dstb)
        pltpu.async_copy(u2d.at[srcb.at[0]], b0, g0)
        pltpu.async_copy(u2d.at[srcb.at[1]], b1, g1)

        def pair(p, c2):
            for j in range(2):
                bb, gg, ss = (b0, g0, s0) if j == 0 else (b1, g1, s1)
                k = p * 2 + j
                pltpu.make_async_copy(u2d.at[srcb.at[k]], bb, gg).wait()
                pltpu.async_copy(bb, acc.at[dstb.at[k]], ss, add=True)

                @pl.when(k + 2 < BST)
                def _():
                    pltpu.make_async_copy(bb, acc.at[dstb.at[k]], ss).wait()
                    pltpu.async_copy(u2d.at[srcb.at[k + 2]], bb, gg)
            return c2

        lax.fori_loop(0, BST // 2, pair, 0)
        pltpu.make_async_copy(b0, acc.at[dstb.at[BST - 2]], s0).wait()
        pltpu.make_async_copy(b1, acc.at[dstb.at[BST - 1]], s1).wait()
        return carry

    lax.fori_loop(0, CW // BST, batch, 0)


_PROP_SCRATCH = [
    pltpu.VMEM_SHARED((NACC, F), jnp.float32),
    pltpu.VMEM((BST, L), jnp.int32),
    pltpu.VMEM((BST, L), jnp.int32),
    pltpu.VMEM((L, F), jnp.float32),
    pltpu.VMEM((L, F), jnp.float32),
    pltpu.SemaphoreType.DMA,
    pltpu.SemaphoreType.DMA,
    pltpu.SemaphoreType.DMA,
    pltpu.SemaphoreType.DMA,
]


def _prop_b_body(u_hbm, ei_hbm, out_hbm,
                 acc, srcb, dstb, b0, b1, g0, g1, s0, s1):
    c = lax.axis_index("c")
    s = lax.axis_index("s")
    _zero_acc(s, acc, b0)
    plsc.subcore_barrier()
    _edge_loop((c * NS + s) * CW, ei_hbm, u_hbm, srcb, dstb, acc,
               b0, b1, g0, g1, s0, s1)
    plsc.subcore_barrier()
    pltpu.sync_copy(acc.at[pl.ds(s * DR, DR)], out_hbm.at[c, pl.ds(s * DR, DR)])


_sc_prop_b = pl.kernel(
    _prop_b_body,
    out_type=jax.ShapeDtypeStruct((NC, NACC, F), jnp.float32),
    mesh=_MESH,
    scratch_types=list(_PROP_SCRATCH),
)


def _prop_c_body(u_hbm, ei_hbm, out_hbm,
                 acc, srcb, dstb, b0, b1, g0, g1, s0, s1):
    c = lax.axis_index("c")
    s = lax.axis_index("s")
    for h in range(2):
        _zero_acc(s, acc, b0)
        plsc.subcore_barrier()
        _edge_loop((c * NS + s) * CW, ei_hbm, u_hbm.at[h], srcb, dstb, acc,
                   b0, b1, g0, g1, s0, s1)
        plsc.subcore_barrier()
        pltpu.sync_copy(
            acc.at[pl.ds(s * DR, DR)], out_hbm.at[h, c, pl.ds(s * DR, DR)])


_sc_prop_c = pl.kernel(
    _prop_c_body,
    out_type=jax.ShapeDtypeStruct((2, NC, NACC, F), jnp.float32),
    mesh=_MESH,
    scratch_types=list(_PROP_SCRATCH),
)


_BLK = 1000
_GRID = N // _BLK


def _tc1_body(x_ref, hist_ref, w1_ref, b1_ref,
              y1_ref, u1_ref, invs_ref, invd_ref):
    cnt = hist_ref[0] + hist_ref[1]
    deg = cnt + 1.0
    invs = lax.rsqrt(deg)
    invd = 1.0 / deg
    y1 = jnp.dot(x_ref[...], w1_ref[...],
                 preferred_element_type=jnp.float32) + b1_ref[...]
    y1_ref[...] = y1
    u1_ref[...] = invs * y1
    invs_ref[...] = invs
    invd_ref[...] = invd


def _tc2_body(mp_ref, y1_ref, invs_ref, invd_ref, w2_ref, b2_ref,
              y2_ref, u2p_ref):
    invs = invs_ref[...]
    invd = invd_ref[...]
    y1 = y1_ref[...]
    m = invs * (mp_ref[0] + mp_ref[1])
    h_lp = jnp.maximum(m + invd * y1, 0.0)
    h_hp = jnp.maximum(y1 - m, 0.0)
    w2 = w2_ref[...]
    b2 = b2_ref[...]
    y2_lp = jnp.dot(h_lp, w2, preferred_element_type=jnp.float32) + b2
    y2_hp = jnp.dot(h_hp, w2, preferred_element_type=jnp.float32) + b2
    y2_ref[0] = y2_lp
    y2_ref[1] = y2_hp
    u2_lp = invs * y2_lp
    u2_hp = invs * y2_hp
    u2p_ref[0] = jnp.concatenate([u2_lp[:, :FH], u2_hp[:, :FH]], axis=-1)
    u2p_ref[1] = jnp.concatenate([u2_lp[:, FH:], u2_hp[:, FH:]], axis=-1)


def _tc3_body(zp_ref, y2_ref, invs_ref, invd_ref, w3_ref, b3_ref,
              zlp_ref, zhp_ref, plp_ref, php_ref):
    invs = invs_ref[...]
    invd = invd_ref[...]
    ph0 = zp_ref[0, 0] + zp_ref[0, 1]
    ph1 = zp_ref[1, 0] + zp_ref[1, 1]
    a_lp = jnp.concatenate([ph0[:, :FH], ph1[:, :FH]], axis=-1)
    a_hp = jnp.concatenate([ph0[:, FH:], ph1[:, FH:]], axis=-1)
    z_lp = invs * a_lp + invd * y2_ref[0]
    z_hp = y2_ref[1] - invs * a_hp
    w3 = w3_ref[...]
    b3 = b3_ref[...]
    zlp_ref[...] = z_lp
    zhp_ref[...] = z_hp
    plp_ref[...] = jnp.dot(z_lp, w3, preferred_element_type=jnp.float32) + b3
    php_ref[...] = jnp.dot(z_hp, w3, preferred_element_type=jnp.float32) + b3


def _rows(i):
    return (i, 0)


def _full(i):
    return (0, 0)


def _rows3(i):
    return (0, i, 0)


def _rows4(i):
    return (0, 0, i, 0)


_spec_nf = pl.BlockSpec((_BLK, F), _rows)
_spec_n1 = pl.BlockSpec((_BLK, 1), _rows)
_spec_2n1 = pl.BlockSpec((NC, _BLK, 1), _rows3)
_spec_2nf = pl.BlockSpec((NC, _BLK, F), _rows3)
_spec_4nf = pl.BlockSpec((2, NC, _BLK, F), _rows4)
_spec_w = pl.BlockSpec((F, F), _full)
_spec_b = pl.BlockSpec((1, F), _full)

_nf = jax.ShapeDtypeStruct((N, F), jnp.float32)
_n1 = jax.ShapeDtypeStruct((N, 1), jnp.float32)
_2nf = jax.ShapeDtypeStruct((NC, N, F), jnp.float32)

_tc1 = pl.pallas_call(
    _tc1_body,
    grid=(_GRID,),
    in_specs=[_spec_nf, _spec_2n1, _spec_w, _spec_b],
    out_specs=[_spec_nf, _spec_nf, _spec_n1, _spec_n1],
    out_shape=[_nf, _nf, _n1, _n1],
)

_tc2 = pl.pallas_call(
    _tc2_body,
    grid=(_GRID,),
    in_specs=[_spec_2nf, _spec_nf, _spec_n1, _spec_n1, _spec_w, _spec_b],
    out_specs=[_spec_2nf, _spec_2nf],
    out_shape=[_2nf, _2nf],
)

_tc3 = pl.pallas_call(
    _tc3_body,
    grid=(_GRID,),
    in_specs=[_spec_4nf, _spec_2nf, _spec_n1, _spec_n1, _spec_w, _spec_b],
    out_specs=[_spec_nf, _spec_nf, _spec_nf, _spec_nf],
    out_shape=[_nf, _nf, _nf, _nf],
)


@jax.jit
def kernel(x, edge_index, W1, b1, W2, b2, W3, b3):
    pad = RT * L - E
    ip = jnp.arange(pad, dtype=jnp.int32)
    src = jnp.concatenate([edge_index[0], N + (ip % Z)])
    dst = jnp.concatenate(
        [edge_index[1], (ip % NS) * DR + (ip // NS) % DR])
    ei = jnp.stack([src, dst]).reshape(2, RT, L)
    dst_hist = jnp.concatenate(
        [edge_index[1], N + (ip % (NP - N))]).reshape(RT, L)

    histp = _sc_hist(dst_hist)
    hist2 = histp[:, :N].reshape(NC, N, 1)

    b1r = b1.reshape(1, F)
    b2r = b2.reshape(1, F)
    b3r = b3.reshape(1, F)

    y1, u1, invs, invd = _tc1(x, hist2, W1, b1r)
    u1z = jnp.concatenate([u1, jnp.zeros((Z, F), jnp.float32)])
    mp = _sc_prop_b(u1z, ei)
    y2, u2p = _tc2(mp, y1, invs, invd, W2, b2r)
    u2pz = jnp.concatenate(
        [u2p, jnp.zeros((NC, Z, F), jnp.float32)], axis=1)
    zp = _sc_prop_c(u2pz, ei)
    z_lp, z_hp, p_lp, p_hp = _tc3(zp, y2, invs, invd, W3, b3r)
    return (z_lp, z_hp, p_lp, p_hp, p_lp, p_hp)
